# Initial kernel scaffold; baseline (speedup 1.0000x reference)
import jax
import jax.numpy as jnp
from jax.experimental import pallas as pl


def kernel(in_features, reduce_index, gather_index, W, b):
    raise NotImplementedError("write your pallas kernel here")


# speedup vs baseline: 7.7511x; 7.7511x over previous
import functools

import jax
import jax.numpy as jnp
from jax import lax
from jax.experimental import pallas as pl
from jax.experimental.pallas import tpu as pltpu
from jax.experimental.pallas import tpu_sc as plsc

N_NODES = 10000
N_EDGES = 320000
C_IN = 128
C_OUT = 128

D = 144
K = 128
NW = 32
CHUNKS = 79
E_PAD = NW * CHUNKS * K
N_PAD = 10240
RPT = N_PAD // 16
TRASH = N_NODES


def _proj_body(x_ref, w_ref, a_ref, bm_ref):
    x = x_ref[...]
    w = w_ref[...]
    w1 = w[:, :C_IN]
    w2 = w[:, C_IN:]
    dn = (((0,), (1,)), ((), ()))
    a_ref[...] = lax.dot_general(x, w1 - w2, dn, preferred_element_type=jnp.float32)
    bm = lax.dot_general(x, w2, dn, preferred_element_type=jnp.float32)
    bn = bm.shape[0]
    ones = jnp.ones((bn, 1), jnp.float32)
    zeros = jnp.zeros((bn, D - C_OUT - 1), jnp.float32)
    bm_ref[...] = jnp.concatenate([bm, ones, zeros], axis=1)


_BN1 = 2048

_proj = pl.pallas_call(
    _proj_body,
    grid=(N_PAD // _BN1,),
    in_specs=[
        pl.BlockSpec((C_IN, _BN1), lambda i: (0, i)),
        pl.BlockSpec((C_OUT, 2 * C_IN), lambda i: (0, 0)),
    ],
    out_specs=[
        pl.BlockSpec((_BN1, C_OUT), lambda i: (i, 0)),
        pl.BlockSpec((_BN1, D), lambda i: (i, 0)),
    ],
    out_shape=[
        jax.ShapeDtypeStruct((N_PAD, C_OUT), jnp.float32),
        jax.ShapeDtypeStruct((N_PAD, D), jnp.float32),
    ],
)


def _sc_body(table, g_hbm, r_hbm, z_hbm, out, g_v, r_v, rows_v, acc, sem):
    cid = lax.axis_index("c")
    sid = lax.axis_index("s")
    row0 = sid * RPT
    pltpu.sync_copy(z_hbm, acc.at[pl.ds(row0, RPT)])
    wid = sid * 2 + cid
    pltpu.sync_copy(g_hbm.at[wid], g_v)
    pltpu.sync_copy(r_hbm.at[wid], r_v)
    plsc.subcore_barrier()

    def body(j, carry):
        pltpu.async_copy(table.at[g_v.at[j]], rows_v, sem).wait()
        pltpu.sync_copy(rows_v, acc.at[r_v.at[j]], add=True)
        return carry

    lax.fori_loop(0, CHUNKS, body, 0)
    plsc.subcore_barrier()
    pltpu.sync_copy(acc.at[pl.ds(row0, RPT)], out.at[cid, pl.ds(row0, RPT)])


@functools.cache
def _sc_scatter():
    return pl.kernel(
        _sc_body,
        mesh=plsc.VectorSubcoreMesh(core_axis_name="c", subcore_axis_name="s"),
        compiler_params=pltpu.CompilerParams(use_tc_tiling_on_sc=False),
        out_type=jax.ShapeDtypeStruct((2, N_PAD, D), jnp.float32),
        scratch_types=[
            pltpu.VMEM((CHUNKS, K), jnp.int32),
            pltpu.VMEM((CHUNKS, K), jnp.int32),
            pltpu.VMEM((K, D), jnp.float32),
            pltpu.VMEM_SHARED((N_PAD, D), jnp.float32),
            pltpu.SemaphoreType.DMA,
        ],
    )


def _comb_body(a_ref, s_ref, b_ref, o_ref):
    s = s_ref[0] + s_ref[1]
    sums = s[:, :C_OUT]
    cnt = s[:, C_OUT:C_OUT + 1]
    val = a_ref[...] + b_ref[...] + sums / jnp.maximum(cnt, 1.0)
    val = jnp.where(cnt > 0, val, 0.0)
    o_ref[...] = jnp.where(val > 0, val, 0.3 * val)


_BN3 = 2048

_comb = pl.pallas_call(
    _comb_body,
    grid=(N_PAD // _BN3,),
    in_specs=[
        pl.BlockSpec((_BN3, C_OUT), lambda i: (i, 0)),
        pl.BlockSpec((2, _BN3, D), lambda i: (0, i, 0)),
        pl.BlockSpec((1, C_OUT), lambda i: (0, 0)),
    ],
    out_specs=pl.BlockSpec((_BN3, C_OUT), lambda i: (i, 0)),
    out_shape=jax.ShapeDtypeStruct((N_PAD, C_OUT), jnp.float32),
)


def kernel(in_features, reduce_index, gather_index, W, b):
    x = in_features[0]
    x_pad = jnp.pad(x, ((0, 0), (0, N_PAD - N_NODES)))
    pad = jnp.full((E_PAD - N_EDGES,), TRASH, jnp.int32)
    g_idx = jnp.concatenate([gather_index, pad]).reshape(NW, CHUNKS, K)
    r_idx = jnp.concatenate([reduce_index, pad]).reshape(NW, CHUNKS, K)
    zeros = jnp.zeros((RPT, D), jnp.float32)

    a_t, table = _proj(x_pad, W)
    partials = _sc_scatter()(table, g_idx, r_idx, zeros)
    out_t = _comb(a_t, partials, b.reshape(1, C_OUT))
    return jnp.transpose(out_t[:N_NODES])[None]
